# baseline (speedup 1.0000x reference)
import functools

import jax
import jax.numpy as jnp
from jax import lax
from jax.experimental import pallas as pl
from jax.experimental.pallas import tpu as pltpu
from jax.experimental.pallas import tpu_sc as plsc

N = 10000
NPAD = 10240
E = 160000
D = 256
RD = D // 2
L = 3
EPS = 1e-5

CH = 128
CHUNKS = E // CH
NW = 32

_mesh = plsc.VectorSubcoreMesh(core_axis_name="c", subcore_axis_name="s")
_CP = pltpu.CompilerParams(needs_layout_passes=False,
                           use_tc_tiling_on_sc=False)


@functools.partial(
    pl.kernel,
    out_type=jax.ShapeDtypeStruct((2 * NW, NPAD), jnp.float32),
    mesh=_mesh,
    compiler_params=_CP,
    scratch_types=[
        pltpu.VMEM((CH,), jnp.int32),
        pltpu.VMEM((CH,), jnp.int32),
        pltpu.VMEM((NPAD,), jnp.float32),
        pltpu.VMEM((NPAD,), jnp.float32),
    ],
)
def _deg(src_hbm, dst_hbm, out_hbm, idxs_v, idxd_v, od_v, id_v):
    c = lax.axis_index("c")
    s = lax.axis_index("s")
    w = s * 2 + c
    zero = jnp.zeros((16,), jnp.float32)
    one = zero + 1.0

    def zbody(i, carry):
        od_v[pl.ds(i * 16, 16)] = zero
        id_v[pl.ds(i * 16, 16)] = zero
        return carry
    lax.fori_loop(0, NPAD // 16, zbody, 0)

    def count_chunk(base):
        pltpu.sync_copy(src_hbm.at[pl.ds(base, CH)], idxs_v)
        pltpu.sync_copy(dst_hbm.at[pl.ds(base, CH)], idxd_v)
        for j in range(CH // 16):
            sv = idxs_v[pl.ds(j * 16, 16)]
            dv = idxd_v[pl.ds(j * 16, 16)]
            plsc.addupdate_scatter(od_v, [sv], one)
            plsc.addupdate_scatter(id_v, [dv], one)

    def body(k, carry):
        count_chunk((w + k * NW) * CH)
        return carry
    lax.fori_loop(0, CHUNKS // NW, body, 0)

    @pl.when(w < CHUNKS - (CHUNKS // NW) * NW)
    def _():
        count_chunk(((CHUNKS // NW) * NW + w) * CH)

    pltpu.sync_copy(od_v, out_hbm.at[w])
    pltpu.sync_copy(id_v, out_hbm.at[NW + w])


@functools.partial(
    pl.kernel,
    out_type=jax.ShapeDtypeStruct((NW, NPAD * 8), jnp.float32),
    mesh=_mesh,
    compiler_params=_CP,
    scratch_types=[
        pltpu.VMEM((CH,), jnp.int32),
        pltpu.VMEM((CH,), jnp.int32),
        pltpu.VMEM((CH, 16), jnp.float32),
        pltpu.VMEM((NPAD * 8,), jnp.float32),
        pltpu.SemaphoreType.DMA,
    ],
)
def _agg(mv_hbm, src_hbm, dst_hbm, out_hbm, idxs_v, idxd_v, rows_v, acc_v,
         sem):
    c = lax.axis_index("c")
    s = lax.axis_index("s")
    w = s * 2 + c
    zero = jnp.zeros((16,), jnp.float32)
    iota = lax.iota(jnp.int32, 16)
    off = iota - c * 8
    msk = (iota >= c * 8) & (iota < c * 8 + 8)

    def zbody(i, carry):
        acc_v[pl.ds(i * 16, 16)] = zero
        return carry
    lax.fori_loop(0, NPAD * 8 // 16, zbody, 0)

    def body(g, carry):
        base = g * CH
        pltpu.sync_copy(src_hbm.at[pl.ds(base, CH)], idxs_v)
        pltpu.sync_copy(dst_hbm.at[pl.ds(base, CH)], idxd_v)
        for j in range(CH // 16):
            sv = idxs_v[pl.ds(j * 16, 16)]
            idxs_v[pl.ds(j * 16, 16)] = sv * 16 + s
        pltpu.async_copy(mv_hbm.at[idxs_v], rows_v, sem).wait()
        for j in range(CH // 16):
            d8 = idxd_v[pl.ds(j * 16, 16)] * 8
            for t in range(16):
                e = j * 16 + t
                rowb = d8[iota * 0 + t]
                val = rows_v[e, :]
                plsc.addupdate_scatter(acc_v, [rowb + off], val, mask=msk)
        return carry
    lax.fori_loop(0, CHUNKS, body, 0)

    pltpu.sync_copy(acc_v, out_hbm.at[w])


def _leaky(v):
    return jnp.where(v >= 0, v, 0.01 * v)


def _prep_body(x_ref, degs_ref, ns_ref, nd_ref, m_ref):
    degs = degs_ref[...]
    ones = jnp.ones((NW, 1), jnp.float32)
    dn = (((0,), (0,)), ((), ()))
    od = lax.dot_general(degs[:NW], ones, dn,
                         preferred_element_type=jnp.float32)
    idg = lax.dot_general(degs[NW:], ones, dn,
                          preferred_element_type=jnp.float32)
    ns = lax.rsqrt(jnp.maximum(od, 1.0))
    nd = lax.rsqrt(jnp.maximum(idg, 1.0))
    ns_ref[...] = ns
    nd_ref[...] = nd
    m_ref[...] = x_ref[...] * ns


def _prep(x, degs):
    return pl.pallas_call(
        _prep_body,
        out_shape=(
            jax.ShapeDtypeStruct((NPAD, 1), jnp.float32),
            jax.ShapeDtypeStruct((NPAD, 1), jnp.float32),
            jax.ShapeDtypeStruct((NPAD, D), jnp.float32),
        ),
    )(x, degs)


def _layer_body(z_ref, nd_ref, ns_ref, w_ref, gnw_ref, gnb_ref, gna_ref,
                pw_ref, pb_ref, rw_ref, rb_ref, r_ref, m_ref):
    z = z_ref[...] * nd_ref[...]
    h = jnp.dot(z, w_ref[...], preferred_element_type=jnp.float32)
    s1 = jnp.sum(h, axis=0, keepdims=True)
    s2 = jnp.sum(h * h, axis=0, keepdims=True)
    mean = s1 * (1.0 / N)
    msq = s2 * (1.0 / N)
    gna = gna_ref[...]
    am = gna * mean
    var = msq - 2.0 * am * mean + am * am
    sub = h - am
    hn = gnw_ref[...] * sub * lax.rsqrt(var + EPS) + gnb_ref[...]
    h2 = _leaky(hn)
    mask = (lax.broadcasted_iota(jnp.int32, (NPAD, 1), 0) < N).astype(
        jnp.float32)
    h2 = h2 * mask
    ph = jnp.maximum(
        jnp.dot(h2, pw_ref[...], preferred_element_type=jnp.float32)
        + pb_ref[...], 0.0)
    sph = jnp.sum(ph * mask, axis=0, keepdims=True)
    r = jnp.dot(sph, rw_ref[...], preferred_element_type=jnp.float32) \
        + rb_ref[...]
    r_ref[...] = _leaky(r)
    m_ref[...] = h2 * ns_ref[...]


def _layer(z, nd, ns, w, gnw, gnb, gna, pw, pb, rw, rb):
    return pl.pallas_call(
        _layer_body,
        out_shape=(
            jax.ShapeDtypeStruct((1, RD), jnp.float32),
            jax.ShapeDtypeStruct((NPAD, D), jnp.float32),
        ),
    )(z, nd, ns, w, gnw, gnb, gna, pw, pb, rw, rb)


def kernel(node_feats, edge_index, Ws, gn_w, gn_b, gn_a, phi_w, phi_b,
           rho_w, rho_b):
    src = edge_index[0]
    dst = edge_index[1]
    x = jnp.pad(node_feats, ((0, NPAD - N), (0, 0)))

    degs = _deg(src, dst)
    ns, nd, m = _prep(x, degs)

    rs = []
    for i in range(L):
        mview = m.reshape(NPAD * 16, 16)
        zrows = _agg(mview, src, dst)
        z = zrows.reshape(NW, NPAD, 8).transpose(1, 0, 2).reshape(NPAD, D)
        r, m = _layer(
            z, nd, ns, Ws[i],
            gn_w[i][None, :], gn_b[i][None, :], gn_a[i][None, :],
            phi_w[i], phi_b[i][None, :], rho_w[i], rho_b[i][None, :])
        rs.append(r)
    return jnp.concatenate(rs, axis=1)


# speedup vs baseline: 1.3585x; 1.3585x over previous
import functools

import jax
import jax.numpy as jnp
from jax import lax
from jax.experimental import pallas as pl
from jax.experimental.pallas import tpu as pltpu
from jax.experimental.pallas import tpu_sc as plsc

N = 10000
NPAD = 10240
E = 160000
D = 256
RD = D // 2
L = 3
EPS = 1e-5

CH = 128
CHUNKS = E // CH
NW = 32

_mesh = plsc.VectorSubcoreMesh(core_axis_name="c", subcore_axis_name="s")
_CP = pltpu.CompilerParams(needs_layout_passes=False,
                           use_tc_tiling_on_sc=False)


@functools.partial(
    pl.kernel,
    out_type=jax.ShapeDtypeStruct((2 * NW, NPAD), jnp.float32),
    mesh=_mesh,
    compiler_params=_CP,
    scratch_types=[
        pltpu.VMEM((CH,), jnp.int32),
        pltpu.VMEM((CH,), jnp.int32),
        pltpu.VMEM((NPAD,), jnp.float32),
        pltpu.VMEM((NPAD,), jnp.float32),
    ],
)
def _deg(src_hbm, dst_hbm, out_hbm, idxs_v, idxd_v, od_v, id_v):
    c = lax.axis_index("c")
    s = lax.axis_index("s")
    w = s * 2 + c
    zero = jnp.zeros((16,), jnp.float32)
    one = zero + 1.0

    def zbody(i, carry):
        od_v[pl.ds(i * 16, 16)] = zero
        id_v[pl.ds(i * 16, 16)] = zero
        return carry
    lax.fori_loop(0, NPAD // 16, zbody, 0)

    def count_chunk(base):
        pltpu.sync_copy(src_hbm.at[pl.ds(base, CH)], idxs_v)
        pltpu.sync_copy(dst_hbm.at[pl.ds(base, CH)], idxd_v)
        for j in range(CH // 16):
            sv = idxs_v[pl.ds(j * 16, 16)]
            dv = idxd_v[pl.ds(j * 16, 16)]
            plsc.addupdate_scatter(od_v, [sv], one)
            plsc.addupdate_scatter(id_v, [dv], one)

    def body(k, carry):
        count_chunk((w + k * NW) * CH)
        return carry
    lax.fori_loop(0, CHUNKS // NW, body, 0)

    @pl.when(w < CHUNKS - (CHUNKS // NW) * NW)
    def _():
        count_chunk(((CHUNKS // NW) * NW + w) * CH)

    pltpu.sync_copy(od_v, out_hbm.at[w])
    pltpu.sync_copy(id_v, out_hbm.at[NW + w])


NBUF = 5
SUPER = CHUNKS // NBUF


@functools.partial(
    pl.kernel,
    out_type=jax.ShapeDtypeStruct((NW, NPAD * 8), jnp.float32),
    mesh=_mesh,
    compiler_params=_CP,
    scratch_types=(
        [pltpu.VMEM((CH,), jnp.int32) for _ in range(NBUF)]
        + [pltpu.VMEM((CH,), jnp.int32) for _ in range(NBUF)]
        + [pltpu.VMEM((CH, 16), jnp.float32) for _ in range(NBUF)]
        + [pltpu.VMEM((NPAD * 8,), jnp.float32)]
        + [pltpu.SemaphoreType.DMA for _ in range(NBUF)]
    ),
)
def _agg(mv_hbm, src_hbm, dst_hbm, out_hbm, *refs):
    idxs_v = refs[0:NBUF]
    idxd_v = refs[NBUF:2 * NBUF]
    rows_v = refs[2 * NBUF:3 * NBUF]
    acc_v = refs[3 * NBUF]
    sems = refs[3 * NBUF + 1:]
    c = lax.axis_index("c")
    s = lax.axis_index("s")
    w = s * 2 + c
    zero = jnp.zeros((16,), jnp.float32)
    iota = lax.iota(jnp.int32, 16)
    off = iota - c * 8
    msk = (iota >= c * 8) & (iota < c * 8 + 8)

    def zbody(i, carry):
        acc_v[pl.ds(i * 16, 16)] = zero
        return carry
    lax.fori_loop(0, NPAD * 8 // 16, zbody, 0)

    def body(g, carry):
        base0 = g * (CH * NBUF)
        for b in range(NBUF):
            base = base0 + b * CH
            pltpu.sync_copy(src_hbm.at[pl.ds(base, CH)], idxs_v[b])
            pltpu.sync_copy(dst_hbm.at[pl.ds(base, CH)], idxd_v[b])
            for j in range(CH // 16):
                sv = idxs_v[b][pl.ds(j * 16, 16)]
                idxs_v[b][pl.ds(j * 16, 16)] = sv * 16 + s
            pltpu.async_copy(mv_hbm.at[idxs_v[b]], rows_v[b], sems[b])
        for b in range(NBUF):
            pltpu.make_async_copy(mv_hbm.at[idxs_v[b]], rows_v[b],
                                  sems[b]).wait()
            for j in range(CH // 16):
                d8 = idxd_v[b][pl.ds(j * 16, 16)] * 8
                for t in range(16):
                    e = j * 16 + t
                    rowb = d8[iota * 0 + t]
                    val = rows_v[b][e, :]
                    plsc.addupdate_scatter(acc_v, [rowb + off], val,
                                           mask=msk)
        return carry
    lax.fori_loop(0, SUPER, body, 0)

    pltpu.sync_copy(acc_v, out_hbm.at[w])


def _leaky(v):
    return jnp.where(v >= 0, v, 0.01 * v)


def _prep_body(x_ref, degs_ref, ns_ref, nd_ref, m_ref):
    degs = degs_ref[...]
    ones = jnp.ones((NW, 1), jnp.float32)
    dn = (((0,), (0,)), ((), ()))
    od = lax.dot_general(degs[:NW], ones, dn,
                         preferred_element_type=jnp.float32)
    idg = lax.dot_general(degs[NW:], ones, dn,
                          preferred_element_type=jnp.float32)
    ns = lax.rsqrt(jnp.maximum(od, 1.0))
    nd = lax.rsqrt(jnp.maximum(idg, 1.0))
    ns_ref[...] = ns
    nd_ref[...] = nd
    m_ref[...] = x_ref[...] * ns


def _prep(x, degs):
    return pl.pallas_call(
        _prep_body,
        out_shape=(
            jax.ShapeDtypeStruct((NPAD, 1), jnp.float32),
            jax.ShapeDtypeStruct((NPAD, 1), jnp.float32),
            jax.ShapeDtypeStruct((NPAD, D), jnp.float32),
        ),
    )(x, degs)


def _layer_body(z_ref, nd_ref, ns_ref, w_ref, gnw_ref, gnb_ref, gna_ref,
                pw_ref, pb_ref, rw_ref, rb_ref, r_ref, m_ref):
    z = z_ref[...] * nd_ref[...]
    h = jnp.dot(z, w_ref[...], preferred_element_type=jnp.float32)
    s1 = jnp.sum(h, axis=0, keepdims=True)
    s2 = jnp.sum(h * h, axis=0, keepdims=True)
    mean = s1 * (1.0 / N)
    msq = s2 * (1.0 / N)
    gna = gna_ref[...]
    am = gna * mean
    var = msq - 2.0 * am * mean + am * am
    sub = h - am
    hn = gnw_ref[...] * sub * lax.rsqrt(var + EPS) + gnb_ref[...]
    h2 = _leaky(hn)
    mask = (lax.broadcasted_iota(jnp.int32, (NPAD, 1), 0) < N).astype(
        jnp.float32)
    h2 = h2 * mask
    ph = jnp.maximum(
        jnp.dot(h2, pw_ref[...], preferred_element_type=jnp.float32)
        + pb_ref[...], 0.0)
    sph = jnp.sum(ph * mask, axis=0, keepdims=True)
    r = jnp.dot(sph, rw_ref[...], preferred_element_type=jnp.float32) \
        + rb_ref[...]
    r_ref[...] = _leaky(r)
    m_ref[...] = h2 * ns_ref[...]


def _layer(z, nd, ns, w, gnw, gnb, gna, pw, pb, rw, rb):
    return pl.pallas_call(
        _layer_body,
        out_shape=(
            jax.ShapeDtypeStruct((1, RD), jnp.float32),
            jax.ShapeDtypeStruct((NPAD, D), jnp.float32),
        ),
    )(z, nd, ns, w, gnw, gnb, gna, pw, pb, rw, rb)


def kernel(node_feats, edge_index, Ws, gn_w, gn_b, gn_a, phi_w, phi_b,
           rho_w, rho_b):
    src = edge_index[0]
    dst = edge_index[1]
    x = jnp.pad(node_feats, ((0, NPAD - N), (0, 0)))

    degs = _deg(src, dst)
    ns, nd, m = _prep(x, degs)

    rs = []
    for i in range(L):
        mview = m.reshape(NPAD * 16, 16)
        zrows = _agg(mview, src, dst)
        z = zrows.reshape(NW, NPAD, 8).transpose(1, 0, 2).reshape(NPAD, D)
        r, m = _layer(
            z, nd, ns, Ws[i],
            gn_w[i][None, :], gn_b[i][None, :], gn_a[i][None, :],
            phi_w[i], phi_b[i][None, :], rho_w[i], rho_b[i][None, :])
        rs.append(r)
    return jnp.concatenate(rs, axis=1)


# speedup vs baseline: 1.4764x; 1.0868x over previous
import functools

import jax
import jax.numpy as jnp
from jax import lax
from jax.experimental import pallas as pl
from jax.experimental.pallas import tpu as pltpu
from jax.experimental.pallas import tpu_sc as plsc

N = 10000
NPAD = 10240
E = 160000
D = 256
RD = D // 2
L = 3
EPS = 1e-5

CH = 128
CHUNKS = E // CH
NW = 32

_mesh = plsc.VectorSubcoreMesh(core_axis_name="c", subcore_axis_name="s")
_CP = pltpu.CompilerParams(needs_layout_passes=False,
                           use_tc_tiling_on_sc=False)


@functools.partial(
    pl.kernel,
    out_type=jax.ShapeDtypeStruct((2 * NW, NPAD), jnp.float32),
    mesh=_mesh,
    compiler_params=_CP,
    scratch_types=[
        pltpu.VMEM((CH,), jnp.int32),
        pltpu.VMEM((CH,), jnp.int32),
        pltpu.VMEM((NPAD,), jnp.float32),
        pltpu.VMEM((NPAD,), jnp.float32),
    ],
)
def _deg(src_hbm, dst_hbm, out_hbm, idxs_v, idxd_v, od_v, id_v):
    c = lax.axis_index("c")
    s = lax.axis_index("s")
    w = s * 2 + c
    zero = jnp.zeros((16,), jnp.float32)
    one = zero + 1.0

    def zbody(i, carry):
        od_v[pl.ds(i * 16, 16)] = zero
        id_v[pl.ds(i * 16, 16)] = zero
        return carry
    lax.fori_loop(0, NPAD // 16, zbody, 0)

    def count_chunk(base):
        pltpu.sync_copy(src_hbm.at[pl.ds(base, CH)], idxs_v)
        pltpu.sync_copy(dst_hbm.at[pl.ds(base, CH)], idxd_v)
        for j in range(CH // 16):
            sv = idxs_v[pl.ds(j * 16, 16)]
            dv = idxd_v[pl.ds(j * 16, 16)]
            plsc.addupdate_scatter(od_v, [sv], one)
            plsc.addupdate_scatter(id_v, [dv], one)

    def body(k, carry):
        count_chunk((w + k * NW) * CH)
        return carry
    lax.fori_loop(0, CHUNKS // NW, body, 0)

    @pl.when(w < CHUNKS - (CHUNKS // NW) * NW)
    def _():
        count_chunk(((CHUNKS // NW) * NW + w) * CH)

    pltpu.sync_copy(od_v, out_hbm.at[w])
    pltpu.sync_copy(id_v, out_hbm.at[NW + w])


NBUF = 5
SUPER = CHUNKS // NBUF


@functools.partial(
    pl.kernel,
    out_type=jax.ShapeDtypeStruct((NW, NPAD * 8), jnp.float32),
    mesh=_mesh,
    compiler_params=_CP,
    scratch_types=(
        [pltpu.VMEM((CH,), jnp.int32) for _ in range(NBUF)]
        + [pltpu.VMEM((CH,), jnp.int32) for _ in range(NBUF)]
        + [pltpu.VMEM((CH, 16), jnp.float32) for _ in range(NBUF)]
        + [pltpu.VMEM((NPAD * 8,), jnp.float32)]
        + [pltpu.SemaphoreType.DMA for _ in range(NBUF)]
    ),
)
def _agg(mv_hbm, src_hbm, dst_hbm, out_hbm, *refs):
    idxs_v = refs[0:NBUF]
    idxd_v = refs[NBUF:2 * NBUF]
    rows_v = refs[2 * NBUF:3 * NBUF]
    acc_v = refs[3 * NBUF]
    sems = refs[3 * NBUF + 1:]
    c = lax.axis_index("c")
    s = lax.axis_index("s")
    w = s * 2 + c
    zero = jnp.zeros((16,), jnp.float32)
    iota = lax.iota(jnp.int32, 16)
    off = iota - c * 8
    msk = (iota >= c * 8) & (iota < c * 8 + 8)

    def zbody(i, carry):
        acc_v[pl.ds(i * 16, 16)] = zero
        return carry
    lax.fori_loop(0, NPAD * 8 // 16, zbody, 0)

    def body(g, carry):
        base0 = g * (CH * NBUF)
        for b in range(NBUF):
            base = base0 + b * CH
            pltpu.sync_copy(src_hbm.at[pl.ds(base, CH)], idxs_v[b])
            pltpu.sync_copy(dst_hbm.at[pl.ds(base, CH)], idxd_v[b])

            def adj(j, carry2, b=b):
                sv = idxs_v[b][pl.ds(j * 16, 16)]
                idxs_v[b][pl.ds(j * 16, 16)] = sv * 16 + s
                return carry2
            lax.fori_loop(0, CH // 16, adj, 0)
            pltpu.async_copy(mv_hbm.at[idxs_v[b]], rows_v[b], sems[b])
        for b in range(NBUF):
            pltpu.make_async_copy(mv_hbm.at[idxs_v[b]], rows_v[b],
                                  sems[b]).wait()

            def scat(j, carry2, b=b):
                d8 = idxd_v[b][pl.ds(j * 16, 16)] * 8
                for t in range(16):
                    rowb = d8[iota * 0 + t]
                    val = rows_v[b][j * 16 + t, :]
                    plsc.addupdate_scatter(acc_v, [rowb + off], val,
                                           mask=msk)
                return carry2
            lax.fori_loop(0, CH // 16, scat, 0)
        return carry
    lax.fori_loop(0, SUPER, body, 0)

    pltpu.sync_copy(acc_v, out_hbm.at[w])


def _leaky(v):
    return jnp.where(v >= 0, v, 0.01 * v)


def _prep_body(x_ref, degs_ref, ns_ref, nd_ref, m_ref):
    degs = degs_ref[...]
    ones = jnp.ones((NW, 1), jnp.float32)
    dn = (((0,), (0,)), ((), ()))
    od = lax.dot_general(degs[:NW], ones, dn,
                         preferred_element_type=jnp.float32)
    idg = lax.dot_general(degs[NW:], ones, dn,
                          preferred_element_type=jnp.float32)
    ns = lax.rsqrt(jnp.maximum(od, 1.0))
    nd = lax.rsqrt(jnp.maximum(idg, 1.0))
    ns_ref[...] = ns
    nd_ref[...] = nd
    m_ref[...] = x_ref[...] * ns


def _prep(x, degs):
    return pl.pallas_call(
        _prep_body,
        out_shape=(
            jax.ShapeDtypeStruct((NPAD, 1), jnp.float32),
            jax.ShapeDtypeStruct((NPAD, 1), jnp.float32),
            jax.ShapeDtypeStruct((NPAD, D), jnp.float32),
        ),
    )(x, degs)


def _layer_body(z_ref, nd_ref, ns_ref, w_ref, gnw_ref, gnb_ref, gna_ref,
                pw_ref, pb_ref, rw_ref, rb_ref, r_ref, m_ref):
    z = z_ref[...] * nd_ref[...]
    h = jnp.dot(z, w_ref[...], preferred_element_type=jnp.float32)
    s1 = jnp.sum(h, axis=0, keepdims=True)
    s2 = jnp.sum(h * h, axis=0, keepdims=True)
    mean = s1 * (1.0 / N)
    msq = s2 * (1.0 / N)
    gna = gna_ref[...]
    am = gna * mean
    var = msq - 2.0 * am * mean + am * am
    sub = h - am
    hn = gnw_ref[...] * sub * lax.rsqrt(var + EPS) + gnb_ref[...]
    h2 = _leaky(hn)
    mask = (lax.broadcasted_iota(jnp.int32, (NPAD, 1), 0) < N).astype(
        jnp.float32)
    h2 = h2 * mask
    ph = jnp.maximum(
        jnp.dot(h2, pw_ref[...], preferred_element_type=jnp.float32)
        + pb_ref[...], 0.0)
    sph = jnp.sum(ph * mask, axis=0, keepdims=True)
    r = jnp.dot(sph, rw_ref[...], preferred_element_type=jnp.float32) \
        + rb_ref[...]
    r_ref[...] = _leaky(r)
    m_ref[...] = h2 * ns_ref[...]


def _layer(z, nd, ns, w, gnw, gnb, gna, pw, pb, rw, rb):
    return pl.pallas_call(
        _layer_body,
        out_shape=(
            jax.ShapeDtypeStruct((1, RD), jnp.float32),
            jax.ShapeDtypeStruct((NPAD, D), jnp.float32),
        ),
    )(z, nd, ns, w, gnw, gnb, gna, pw, pb, rw, rb)


def kernel(node_feats, edge_index, Ws, gn_w, gn_b, gn_a, phi_w, phi_b,
           rho_w, rho_b):
    src = edge_index[0]
    dst = edge_index[1]
    x = jnp.pad(node_feats, ((0, NPAD - N), (0, 0)))

    degs = _deg(src, dst)
    ns, nd, m = _prep(x, degs)

    rs = []
    for i in range(L):
        mview = m.reshape(NPAD * 16, 16)
        zrows = _agg(mview, src, dst)
        z = zrows.reshape(NW, NPAD, 8).transpose(1, 0, 2).reshape(NPAD, D)
        r, m = _layer(
            z, nd, ns, Ws[i],
            gn_w[i][None, :], gn_b[i][None, :], gn_a[i][None, :],
            phi_w[i], phi_b[i][None, :], rho_w[i], rho_b[i][None, :])
        rs.append(r)
    return jnp.concatenate(rs, axis=1)
